# Initial kernel scaffold; baseline (speedup 1.0000x reference)
import jax
import jax.numpy as jnp
from jax.experimental import pallas as pl


def kernel(nodes_embed, edges_embed, edges, W_self, W_agg):
    raise NotImplementedError("write your pallas kernel here")


# speedup vs baseline: 5.0373x; 5.0373x over previous
import functools

import jax
import jax.numpy as jnp
from jax import lax
from jax.experimental import pallas as pl
from jax.experimental.pallas import tpu as pltpu
from jax.experimental.pallas import tpu_sc as plsc

NC = 2
NS = 16
NW = NC * NS
CHUNK = 128
BLK = 1280


def _sc_segment_sum(nodes_pad, edges_embed, src2d, rel2d, dst2d, zrows, zcnt):
  n_pad, d = nodes_pad.shape
  n_chunks = src2d.shape[0]
  slice_rows = n_pad // NS
  mesh = plsc.VectorSubcoreMesh(core_axis_name="c", subcore_axis_name="s")

  @functools.partial(
      pl.kernel,
      out_type=(
          jax.ShapeDtypeStruct((n_pad, d), jnp.float32),
          jax.ShapeDtypeStruct((n_pad, d), jnp.float32),
          jax.ShapeDtypeStruct((n_pad,), jnp.float32),
          jax.ShapeDtypeStruct((n_pad,), jnp.float32),
      ),
      mesh=mesh,
      scratch_types=[
          pltpu.VMEM((CHUNK,), jnp.int32),
          pltpu.VMEM((CHUNK,), jnp.int32),
          pltpu.VMEM((CHUNK,), jnp.int32),
          pltpu.VMEM((CHUNK, d), jnp.float32),
          pltpu.VMEM((CHUNK, d), jnp.float32),
          pltpu.VMEM((CHUNK,), jnp.float32),
          pltpu.VMEM_SHARED((n_pad, d), jnp.float32),
          pltpu.VMEM_SHARED((n_pad,), jnp.float32),
      ],
  )
  def k(nodes_hbm, eemb_hbm, src_hbm, rel_hbm, dst_hbm, zr_hbm, zc_hbm,
        acc0_out, acc1_out, cnt0_out, cnt1_out,
        idx_s, idx_r, idx_d, rows_a, rows_b, ones, acc, cnt):
    c = lax.axis_index("c")
    s = lax.axis_index("s")
    wid = c * NS + s
    rowsl = pl.ds(s * slice_rows, slice_rows)

    pltpu.sync_copy(zr_hbm, acc.at[rowsl])
    pltpu.sync_copy(zc_hbm, cnt.at[rowsl])
    for kk in range(CHUNK // 16):
      ones[pl.ds(kk * 16, 16)] = jnp.ones((16,), jnp.float32)
    plsc.subcore_barrier()

    def body(j, carry):
      chunk = j * NW + wid

      @pl.when(chunk < n_chunks)
      def _():
        pltpu.sync_copy(src_hbm.at[chunk], idx_s)
        pltpu.sync_copy(rel_hbm.at[chunk], idx_r)
        pltpu.sync_copy(dst_hbm.at[chunk], idx_d)
        pltpu.sync_copy(nodes_hbm.at[idx_s], rows_a)
        pltpu.sync_copy(eemb_hbm.at[idx_r], rows_b)
        pltpu.sync_copy(rows_a, acc.at[idx_d], add=True)
        pltpu.sync_copy(rows_b, acc.at[idx_d], add=True)
        pltpu.sync_copy(ones, cnt.at[idx_d], add=True)

      return carry

    lax.fori_loop(0, (n_chunks + NW - 1) // NW, body, 0)
    plsc.subcore_barrier()

    @pl.when(c == 0)
    def _():
      pltpu.sync_copy(acc.at[rowsl], acc0_out.at[rowsl])
      pltpu.sync_copy(cnt.at[rowsl], cnt0_out.at[rowsl])

    @pl.when(c == 1)
    def _():
      pltpu.sync_copy(acc.at[rowsl], acc1_out.at[rowsl])
      pltpu.sync_copy(cnt.at[rowsl], cnt1_out.at[rowsl])

  return k(nodes_pad, edges_embed, src2d, rel2d, dst2d, zrows, zcnt)


def _tc_combine(nodes_pad, w_self, w_agg, p0, p1, c0, c1):
  n_pad, d = nodes_pad.shape

  def body(n_ref, ws_ref, wa_ref, p0_ref, p1_ref, c0_ref, c1_ref, o_ref):
    h = lax.dot_general(n_ref[...], ws_ref[...], (((1,), (1,)), ((), ())),
                        preferred_element_type=jnp.float32)
    p = p0_ref[...] + p1_ref[...]
    m = lax.dot_general(p, wa_ref[...], (((1,), (1,)), ((), ())),
                        preferred_element_type=jnp.float32)
    i = pl.program_id(0)
    cntv = c0_ref[pl.ds(i * BLK, BLK)] + c1_ref[pl.ds(i * BLK, BLK)]
    inv = 1.0 / jnp.maximum(cntv, 1.0)
    o_ref[...] = jnp.tanh(h + m * inv[:, None])

  return pl.pallas_call(
      body,
      grid=(n_pad // BLK,),
      in_specs=[
          pl.BlockSpec((BLK, d), lambda i: (i, 0)),
          pl.BlockSpec((d, d), lambda i: (0, 0)),
          pl.BlockSpec((d, d), lambda i: (0, 0)),
          pl.BlockSpec((BLK, d), lambda i: (i, 0)),
          pl.BlockSpec((BLK, d), lambda i: (i, 0)),
          pl.BlockSpec((n_pad,), lambda i: (0,)),
          pl.BlockSpec((n_pad,), lambda i: (0,)),
      ],
      out_specs=pl.BlockSpec((BLK, d), lambda i: (i, 0)),
      out_shape=jax.ShapeDtypeStruct((n_pad, d), jnp.float32),
  )(nodes_pad, w_self, w_agg, p0, p1, c0, c1)


def kernel(nodes_embed, edges_embed, edges, W_self, W_agg):
  n_nodes, d = nodes_embed.shape
  n_edges = edges.shape[0]
  n_chunks = n_edges // CHUNK
  n_pad = ((n_nodes + BLK - 1) // BLK) * BLK

  src2d = edges[:, 0].reshape(n_chunks, CHUNK)
  rel2d = edges[:, 1].reshape(n_chunks, CHUNK)
  dst2d = edges[:, 2].reshape(n_chunks, CHUNK)
  nodes_pad = jnp.pad(nodes_embed, ((0, n_pad - n_nodes), (0, 0)))
  zrows = jnp.zeros((n_pad // NS, d), jnp.float32)
  zcnt = jnp.zeros((n_pad // NS,), jnp.float32)

  acc0, acc1, cnt0, cnt1 = _sc_segment_sum(
      nodes_pad, edges_embed, src2d, rel2d, dst2d, zrows, zcnt)
  out = _tc_combine(nodes_pad, W_self, W_agg, acc0, acc1, cnt0, cnt1)
  return out[:n_nodes]


# speedup vs baseline: 9.5883x; 1.9035x over previous
import functools

import jax
import jax.numpy as jnp
from jax import lax
from jax.experimental import pallas as pl
from jax.experimental.pallas import tpu as pltpu
from jax.experimental.pallas import tpu_sc as plsc

NC = 2
NS = 16
NW = NC * NS
CHUNK = 80
IDEPTH = 4
BLK = 1280


def _sc_segment_sum(nodes_pad, edges_embed, idx3, zrows, zcnt):
  n_pad, d = nodes_pad.shape
  n_chunks = idx3.shape[0]
  slice_rows = n_pad // NS
  cpt = n_chunks // NW
  mesh = plsc.VectorSubcoreMesh(core_axis_name="c", subcore_axis_name="s")

  @functools.partial(
      pl.kernel,
      out_type=(
          jax.ShapeDtypeStruct((n_pad, d), jnp.float32),
          jax.ShapeDtypeStruct((n_pad, d), jnp.float32),
          jax.ShapeDtypeStruct((n_pad,), jnp.float32),
          jax.ShapeDtypeStruct((n_pad,), jnp.float32),
      ),
      mesh=mesh,
      scratch_types=[
          pltpu.VMEM((IDEPTH, 3, CHUNK), jnp.int32),
          pltpu.VMEM((2, CHUNK, d), jnp.float32),
          pltpu.VMEM((2, CHUNK, d), jnp.float32),
          pltpu.VMEM((CHUNK,), jnp.float32),
          pltpu.VMEM_SHARED((n_pad, d), jnp.float32),
          pltpu.VMEM_SHARED((n_pad,), jnp.float32),
          pltpu.SemaphoreType.DMA((IDEPTH,)),
          pltpu.SemaphoreType.DMA((2,)),
          pltpu.SemaphoreType.DMA((2,)),
          pltpu.SemaphoreType.DMA((2,)),
      ],
  )
  def k(nodes_hbm, eemb_hbm, idx_hbm, zr_hbm, zc_hbm,
        acc0_out, acc1_out, cnt0_out, cnt1_out,
        idx_v, rows_a, rows_b, ones, acc, cnt,
        sem_i, sem_a, sem_b, sem_s):
    c = lax.axis_index("c")
    s = lax.axis_index("s")
    wid = c * NS + s
    rowsl = pl.ds(s * slice_rows, slice_rows)
    base = wid * cpt

    def fire_idx(j, q):
      pltpu.async_copy(idx_hbm.at[base + j], idx_v.at[q], sem_i.at[q])

    def wait_idx(j, q):
      pltpu.make_async_copy(
          idx_hbm.at[base + j], idx_v.at[q], sem_i.at[q]).wait()

    def fire_gather(j, b, q):
      pltpu.async_copy(nodes_hbm.at[idx_v.at[q, 0]], rows_a.at[b], sem_a.at[b])
      pltpu.async_copy(eemb_hbm.at[idx_v.at[q, 1]], rows_b.at[b], sem_b.at[b])

    def wait_gather(j, b, q):
      pltpu.make_async_copy(
          nodes_hbm.at[idx_v.at[q, 0]], rows_a.at[b], sem_a.at[b]).wait()
      pltpu.make_async_copy(
          eemb_hbm.at[idx_v.at[q, 1]], rows_b.at[b], sem_b.at[b]).wait()

    def fire_scatter(j, b, q):
      pltpu.async_copy(rows_a.at[b], acc.at[idx_v.at[q, 2]], sem_s.at[b],
                       add=True)
      pltpu.async_copy(rows_b.at[b], acc.at[idx_v.at[q, 2]], sem_s.at[b],
                       add=True)
      pltpu.async_copy(ones, cnt.at[idx_v.at[q, 2]], sem_s.at[b], add=True)

    def wait_scatter(j, b, q):
      pltpu.make_async_copy(
          rows_a.at[b], acc.at[idx_v.at[q, 2]], sem_s.at[b]).wait()
      pltpu.make_async_copy(
          rows_b.at[b], acc.at[idx_v.at[q, 2]], sem_s.at[b]).wait()
      pltpu.make_async_copy(
          ones, cnt.at[idx_v.at[q, 2]], sem_s.at[b]).wait()

    for j in range(min(IDEPTH - 1, cpt)):
      fire_idx(j, j % IDEPTH)
    pltpu.sync_copy(zr_hbm, acc.at[rowsl])
    pltpu.sync_copy(zc_hbm, cnt.at[rowsl])
    for kk in range(CHUNK // 16):
      ones[pl.ds(kk * 16, 16)] = jnp.ones((16,), jnp.float32)
    plsc.subcore_barrier()

    wait_idx(0, 0)
    fire_gather(0, 0, 0)

    n_outer = (cpt + IDEPTH) // IDEPTH

    def body(jj, carry):
      for u in range(IDEPTH):
        j = jj * IDEPTH + u
        b = u % 2
        q = u % IDEPTH

        @pl.when((j >= 1) & (j <= cpt))
        def _():
          wait_scatter(j - 1, 1 - b, (q - 1) % IDEPTH)

        @pl.when(j + IDEPTH - 1 < cpt)
        def _():
          fire_idx(j + IDEPTH - 1, (q + IDEPTH - 1) % IDEPTH)

        @pl.when(j + 1 < cpt)
        def _():
          wait_idx(j + 1, (q + 1) % IDEPTH)
          fire_gather(j + 1, 1 - b, (q + 1) % IDEPTH)

        @pl.when(j < cpt)
        def _():
          wait_gather(j, b, q)
          fire_scatter(j, b, q)

      return carry

    lax.fori_loop(0, n_outer, body, 0)
    plsc.subcore_barrier()

    @pl.when(c == 0)
    def _():
      pltpu.sync_copy(acc.at[rowsl], acc0_out.at[rowsl])
      pltpu.sync_copy(cnt.at[rowsl], cnt0_out.at[rowsl])

    @pl.when(c == 1)
    def _():
      pltpu.sync_copy(acc.at[rowsl], acc1_out.at[rowsl])
      pltpu.sync_copy(cnt.at[rowsl], cnt1_out.at[rowsl])

  return k(nodes_pad, edges_embed, idx3, zrows, zcnt)


def _tc_combine(nodes_pad, w_self, w_agg, p0, p1, c0, c1):
  n_pad, d = nodes_pad.shape

  def body(n_ref, ws_ref, wa_ref, p0_ref, p1_ref, c0_ref, c1_ref, o_ref):
    h = lax.dot_general(n_ref[...], ws_ref[...], (((1,), (1,)), ((), ())),
                        preferred_element_type=jnp.float32)
    p = p0_ref[...] + p1_ref[...]
    m = lax.dot_general(p, wa_ref[...], (((1,), (1,)), ((), ())),
                        preferred_element_type=jnp.float32)
    i = pl.program_id(0)
    cntv = c0_ref[pl.ds(i * BLK, BLK)] + c1_ref[pl.ds(i * BLK, BLK)]
    inv = 1.0 / jnp.maximum(cntv, 1.0)
    o_ref[...] = jnp.tanh(h + m * inv[:, None])

  return pl.pallas_call(
      body,
      grid=(n_pad // BLK,),
      in_specs=[
          pl.BlockSpec((BLK, d), lambda i: (i, 0)),
          pl.BlockSpec((d, d), lambda i: (0, 0)),
          pl.BlockSpec((d, d), lambda i: (0, 0)),
          pl.BlockSpec((BLK, d), lambda i: (i, 0)),
          pl.BlockSpec((BLK, d), lambda i: (i, 0)),
          pl.BlockSpec((n_pad,), lambda i: (0,)),
          pl.BlockSpec((n_pad,), lambda i: (0,)),
      ],
      out_specs=pl.BlockSpec((BLK, d), lambda i: (i, 0)),
      out_shape=jax.ShapeDtypeStruct((n_pad, d), jnp.float32),
  )(nodes_pad, w_self, w_agg, p0, p1, c0, c1)


def kernel(nodes_embed, edges_embed, edges, W_self, W_agg):
  n_nodes, d = nodes_embed.shape
  n_edges = edges.shape[0]
  n_chunks = n_edges // CHUNK
  n_pad = ((n_nodes + BLK - 1) // BLK) * BLK

  idx3 = jnp.transpose(edges.reshape(n_chunks, CHUNK, 3), (0, 2, 1))
  nodes_pad = jnp.pad(nodes_embed, ((0, n_pad - n_nodes), (0, 0)))
  zrows = jnp.zeros((n_pad // NS, d), jnp.float32)
  zcnt = jnp.zeros((n_pad // NS,), jnp.float32)

  acc0, acc1, cnt0, cnt1 = _sc_segment_sum(
      nodes_pad, edges_embed, idx3, zrows, zcnt)
  out = _tc_combine(nodes_pad, W_self, W_agg, acc0, acc1, cnt0, cnt1)
  return out[:n_nodes]
